# baseline (speedup 1.0000x reference)
import jax
import jax.numpy as jnp
from jax import lax
from jax.experimental import pallas as pl
from jax.experimental.pallas import tpu as pltpu
from jax.experimental.pallas import tpu_sc as plsc

N = 10000
D = 128
E = 320000
NC = 2
NS = 16
NW = NC * NS
L = 16
EPW = E // NW
C = 80
NCHUNK = EPW // C


def _norm_body(mult_ref, x1_ref, x2_ref, o1_ref, o2_ref):
    for x_ref, o_ref, m in ((x1_ref, o1_ref, 0), (x2_ref, o2_ref, 1)):
        x = x_ref[...]
        n2 = jnp.sum(x * x, axis=1, keepdims=True)
        r = jnp.maximum(jnp.sqrt(n2), 1e-8)
        o_ref[...] = x * (mult_ref[0, m] / r)


def _normalize2(t1, t2, scale):
    mult = jnp.stack([jnp.asarray(scale, jnp.float32),
                      jnp.float32(1.0)]).reshape(1, 2)
    blk = pl.BlockSpec((N // 10, D), lambda i: (i, 0))
    return pl.pallas_call(
        _norm_body,
        grid=(10,),
        in_specs=[pl.BlockSpec(memory_space=pltpu.SMEM), blk, blk],
        out_specs=[blk, blk],
        out_shape=[jax.ShapeDtypeStruct((N, D), jnp.float32)] * 2,
    )(mult, t1, t2)


def _sc_body(pn_hbm, cn_hbm, ei_hbm, bias_hbm, out_hbm,
             sidx, didx, srowsA, drowsA, srowsB, drowsB, outv, biasv,
             semA, semB, semI):
    cid = lax.axis_index("c")
    sid = lax.axis_index("s")
    wid = sid * NC + cid
    base = wid * EPW

    pltpu.sync_copy(bias_hbm, biasv)
    c1 = pltpu.async_copy(ei_hbm.at[pl.ds(base, EPW)], sidx, semI)
    c2 = pltpu.async_copy(ei_hbm.at[pl.ds(E + base, EPW)], didx, semI)
    c1.wait()
    c2.wait()
    bvec = biasv[...]

    def _issue(ci, srows, drows, sem):
        off = ci * C
        pltpu.async_copy(pn_hbm.at[sidx.at[pl.ds(off, C)]], srows, sem)
        pltpu.async_copy(cn_hbm.at[didx.at[pl.ds(off, C)]], drows, sem)

    def _wait(srows, drows, sem):
        pltpu.make_async_copy(pn_hbm.at[pl.ds(0, C)], srows, sem).wait()
        pltpu.make_async_copy(cn_hbm.at[pl.ds(0, C)], drows, sem).wait()

    def _compute(ci, srows, drows):
        lane = lax.iota(jnp.int32, L)

        @pl.loop(0, C // L)
        def _grp(g):
            rows = g * L + lane

            def _k(k, acc):
                kv = (lane + k) & (D - 1)
                sv = plsc.load_gather(srows, [rows, kv])
                dv = plsc.load_gather(drows, [rows, kv])
                return acc + sv * dv

            acc = lax.fori_loop(0, D, _k, jnp.zeros((L,), jnp.float32),
                                unroll=8)
            outv[pl.ds(ci * C + g * L, L)] = acc + bvec

    _issue(0, srowsA, drowsA, semA)

    @pl.loop(0, (NCHUNK - 1) // 2)
    def _pair(i):
        ci = 1 + 2 * i
        _issue(ci, srowsB, drowsB, semB)
        _wait(srowsA, drowsA, semA)
        _compute(ci - 1, srowsA, drowsA)
        _issue(ci + 1, srowsA, drowsA, semA)
        _wait(srowsB, drowsB, semB)
        _compute(ci, srowsB, drowsB)

    _wait(srowsA, drowsA, semA)
    _compute(NCHUNK - 1, srowsA, drowsA)

    pltpu.sync_copy(outv, out_hbm.at[pl.ds(base, EPW)])


def _sc_call(pn, cn, ei, bias16):
    mesh = plsc.VectorSubcoreMesh(core_axis_name="c", subcore_axis_name="s",
                                  num_cores=NC, num_subcores=NS)
    f = pl.kernel(
        _sc_body,
        out_type=jax.ShapeDtypeStruct((E,), jnp.float32),
        mesh=mesh,
        compiler_params=pltpu.CompilerParams(needs_layout_passes=False),
        scratch_types=[
            pltpu.VMEM((EPW,), jnp.int32),
            pltpu.VMEM((EPW,), jnp.int32),
            pltpu.VMEM((C, D), jnp.float32),
            pltpu.VMEM((C, D), jnp.float32),
            pltpu.VMEM((C, D), jnp.float32),
            pltpu.VMEM((C, D), jnp.float32),
            pltpu.VMEM((EPW,), jnp.float32),
            pltpu.VMEM((L,), jnp.float32),
            pltpu.SemaphoreType.DMA,
            pltpu.SemaphoreType.DMA,
            pltpu.SemaphoreType.DMA,
        ],
    )
    return f(pn, cn, ei, bias16)


def kernel(patient_embeds, condition_embeds, edge_index, scale, bias):
    pn, cn = _normalize2(patient_embeds, condition_embeds, scale)
    ei = edge_index.astype(jnp.int32).reshape(2 * E)
    bias16 = jnp.broadcast_to(bias.astype(jnp.float32), (16,))
    return _sc_call(pn, cn, ei, bias16)


# speedup vs baseline: 1.0000x; 1.0000x over previous
import jax
import jax.numpy as jnp
from jax import lax
from jax.experimental import pallas as pl
from jax.experimental.pallas import tpu as pltpu
from jax.experimental.pallas import tpu_sc as plsc

N = 10000
D = 128
E = 320000
NC = 2
NS = 16
NW = NC * NS
L = 16
EPW = E // NW
C = 80
NCHUNK = EPW // C


def _norm_body(mult_ref, x1_ref, x2_ref, o1_ref, o2_ref):
    for x_ref, o_ref, m in ((x1_ref, o1_ref, 0), (x2_ref, o2_ref, 1)):
        x = x_ref[...]
        n2 = jnp.sum(x * x, axis=1, keepdims=True)
        r = jnp.maximum(jnp.sqrt(n2), 1e-8)
        o_ref[...] = x * (mult_ref[0, m] / r)


def _normalize2(t1, t2, scale):
    mult = jnp.stack([jnp.asarray(scale, jnp.float32),
                      jnp.float32(1.0)]).reshape(1, 2)
    blk = pl.BlockSpec((N // 10, D), lambda i: (i, 0))
    return pl.pallas_call(
        _norm_body,
        grid=(10,),
        in_specs=[pl.BlockSpec(memory_space=pltpu.SMEM), blk, blk],
        out_specs=[blk, blk],
        out_shape=[jax.ShapeDtypeStruct((N, D), jnp.float32)] * 2,
    )(mult, t1, t2)


def _sc_body(pn_hbm, cn_hbm, ei_hbm, bias_hbm, out_hbm,
             sidx, didx, srowsA, drowsA, srowsB, drowsB, outv, biasv,
             semA, semB, semI):
    cid = lax.axis_index("c")
    sid = lax.axis_index("s")
    wid = sid * NC + cid
    base = wid * EPW

    pltpu.sync_copy(bias_hbm, biasv)
    c1 = pltpu.async_copy(ei_hbm.at[pl.ds(base, EPW)], sidx, semI)
    c2 = pltpu.async_copy(ei_hbm.at[pl.ds(E + base, EPW)], didx, semI)
    c1.wait()
    c2.wait()
    bvec = biasv[...]

    def _issue(ci, srows, drows, sem):
        off = ci * C
        pltpu.async_copy(pn_hbm.at[sidx.at[pl.ds(off, C)]], srows, sem)
        pltpu.async_copy(cn_hbm.at[didx.at[pl.ds(off, C)]], drows, sem)

    def _wait(srows, drows, sem):
        pltpu.make_async_copy(pn_hbm.at[pl.ds(0, C)], srows, sem).wait()
        pltpu.make_async_copy(cn_hbm.at[pl.ds(0, C)], drows, sem).wait()

    def _compute(ci, srows, drows):
        lane = lax.iota(jnp.int32, L)

        @pl.loop(0, C // L)
        def _grp(g):
            rows = g * L + lane

            def _k(k, acc):
                kv = (lane + k) & (D - 1)
                sv = plsc.load_gather(srows, [rows, kv])
                dv = plsc.load_gather(drows, [rows, kv])
                return acc + sv * dv

            acc = lax.fori_loop(0, D, _k, jnp.zeros((L,), jnp.float32),
                                unroll=16)
            outv[pl.ds(ci * C + g * L, L)] = acc + bvec

    _issue(0, srowsA, drowsA, semA)

    @pl.loop(0, (NCHUNK - 1) // 2)
    def _pair(i):
        ci = 1 + 2 * i
        _issue(ci, srowsB, drowsB, semB)
        _wait(srowsA, drowsA, semA)
        _compute(ci - 1, srowsA, drowsA)
        _issue(ci + 1, srowsA, drowsA, semA)
        _wait(srowsB, drowsB, semB)
        _compute(ci, srowsB, drowsB)

    _wait(srowsA, drowsA, semA)
    _compute(NCHUNK - 1, srowsA, drowsA)

    pltpu.sync_copy(outv, out_hbm.at[pl.ds(base, EPW)])


def _sc_call(pn, cn, ei, bias16):
    mesh = plsc.VectorSubcoreMesh(core_axis_name="c", subcore_axis_name="s",
                                  num_cores=NC, num_subcores=NS)
    f = pl.kernel(
        _sc_body,
        out_type=jax.ShapeDtypeStruct((E,), jnp.float32),
        mesh=mesh,
        compiler_params=pltpu.CompilerParams(needs_layout_passes=False),
        scratch_types=[
            pltpu.VMEM((EPW,), jnp.int32),
            pltpu.VMEM((EPW,), jnp.int32),
            pltpu.VMEM((C, D), jnp.float32),
            pltpu.VMEM((C, D), jnp.float32),
            pltpu.VMEM((C, D), jnp.float32),
            pltpu.VMEM((C, D), jnp.float32),
            pltpu.VMEM((EPW,), jnp.float32),
            pltpu.VMEM((L,), jnp.float32),
            pltpu.SemaphoreType.DMA,
            pltpu.SemaphoreType.DMA,
            pltpu.SemaphoreType.DMA,
        ],
    )
    return f(pn, cn, ei, bias16)


def kernel(patient_embeds, condition_embeds, edge_index, scale, bias):
    pn, cn = _normalize2(patient_embeds, condition_embeds, scale)
    ei = edge_index.astype(jnp.int32).reshape(2 * E)
    bias16 = jnp.broadcast_to(bias.astype(jnp.float32), (16,))
    return _sc_call(pn, cn, ei, bias16)


# speedup vs baseline: 1.0905x; 1.0904x over previous
import jax
import jax.numpy as jnp
from jax import lax
from jax.experimental import pallas as pl
from jax.experimental.pallas import tpu as pltpu
from jax.experimental.pallas import tpu_sc as plsc

N = 10000
D = 128
E = 320000
NC = 2
NS = 16
NW = NC * NS
L = 16
EPW = E // NW
C = 128
NCHUNK = EPW // C
TAIL = EPW - NCHUNK * C


def _norm_body(mult_ref, x1_ref, x2_ref, o1_ref, o2_ref):
    for x_ref, o_ref, m in ((x1_ref, o1_ref, 0), (x2_ref, o2_ref, 1)):
        x = x_ref[...]
        n2 = jnp.sum(x * x, axis=1, keepdims=True)
        r = jnp.maximum(jnp.sqrt(n2), 1e-8)
        o_ref[...] = x * (mult_ref[0, m] / r)


def _normalize2(t1, t2, scale):
    mult = jnp.stack([jnp.asarray(scale, jnp.float32),
                      jnp.float32(1.0)]).reshape(1, 2)
    blk = pl.BlockSpec((N // 10, D), lambda i: (i, 0))
    return pl.pallas_call(
        _norm_body,
        grid=(10,),
        in_specs=[pl.BlockSpec(memory_space=pltpu.SMEM), blk, blk],
        out_specs=[blk, blk],
        out_shape=[jax.ShapeDtypeStruct((N, D), jnp.float32)] * 2,
    )(mult, t1, t2)


def _sc_body(pn_hbm, cn_hbm, ei_hbm, bias_hbm, out_hbm,
             sidx, didx, srowsA, drowsA, srowsB, drowsB, outv, biasv,
             semA, semB, semI):
    cid = lax.axis_index("c")
    sid = lax.axis_index("s")
    wid = sid * NC + cid
    base = wid * EPW

    pltpu.sync_copy(bias_hbm, biasv)
    c1 = pltpu.async_copy(ei_hbm.at[pl.ds(base, EPW)], sidx, semI)
    c2 = pltpu.async_copy(ei_hbm.at[pl.ds(E + base, EPW)], didx, semI)
    c1.wait()
    c2.wait()
    bvec = biasv[...]

    def _issue(ci, srows, drows, sem):
        off = ci * C
        pltpu.async_copy(pn_hbm.at[sidx.at[pl.ds(off, C)]], srows, sem)
        pltpu.async_copy(cn_hbm.at[didx.at[pl.ds(off, C)]], drows, sem)

    def _wait(srows, drows, sem):
        pltpu.make_async_copy(pn_hbm.at[pl.ds(0, C)], srows, sem).wait()
        pltpu.make_async_copy(cn_hbm.at[pl.ds(0, C)], drows, sem).wait()

    lane = lax.iota(jnp.int32, L)

    def _dot_group(srows, drows, rows):
        def _k(k, acc):
            kv = (lane + k) & (D - 1)
            sv = plsc.load_gather(srows, [rows, kv])
            dv = plsc.load_gather(drows, [rows, kv])
            return acc + sv * dv

        return lax.fori_loop(0, D, _k, jnp.zeros((L,), jnp.float32),
                             unroll=16)

    def _compute(ci, srows, drows):
        @pl.loop(0, C // L)
        def _grp(g):
            acc = _dot_group(srows, drows, g * L + lane)
            outv[pl.ds(ci * C + g * L, L)] = acc + bvec

    _issue(0, srowsA, drowsA, semA)

    @pl.loop(0, (NCHUNK - 2) // 2)
    def _pair(i):
        ci = 1 + 2 * i
        _issue(ci, srowsB, drowsB, semB)
        _wait(srowsA, drowsA, semA)
        _compute(ci - 1, srowsA, drowsA)
        _issue(ci + 1, srowsA, drowsA, semA)
        _wait(srowsB, drowsB, semB)
        _compute(ci, srowsB, drowsB)

    _issue(NCHUNK - 1, srowsB, drowsB, semB)
    _wait(srowsA, drowsA, semA)
    _compute(NCHUNK - 2, srowsA, drowsA)
    toff = NCHUNK * C
    pltpu.async_copy(pn_hbm.at[sidx.at[pl.ds(toff, TAIL)]],
                     srowsA.at[pl.ds(0, TAIL), :], semA)
    pltpu.async_copy(cn_hbm.at[didx.at[pl.ds(toff, TAIL)]],
                     drowsA.at[pl.ds(0, TAIL), :], semA)
    _wait(srowsB, drowsB, semB)
    _compute(NCHUNK - 1, srowsB, drowsB)
    pltpu.make_async_copy(pn_hbm.at[pl.ds(0, TAIL)],
                          srowsA.at[pl.ds(0, TAIL), :], semA).wait()
    pltpu.make_async_copy(cn_hbm.at[pl.ds(0, TAIL)],
                          drowsA.at[pl.ds(0, TAIL), :], semA).wait()
    acc = _dot_group(srowsA, drowsA, lane)
    outv[pl.ds(toff, L)] = acc + bvec

    pltpu.sync_copy(outv, out_hbm.at[pl.ds(base, EPW)])


def _sc_call(pn, cn, ei, bias16):
    mesh = plsc.VectorSubcoreMesh(core_axis_name="c", subcore_axis_name="s",
                                  num_cores=NC, num_subcores=NS)
    f = pl.kernel(
        _sc_body,
        out_type=jax.ShapeDtypeStruct((E,), jnp.float32),
        mesh=mesh,
        compiler_params=pltpu.CompilerParams(needs_layout_passes=False),
        scratch_types=[
            pltpu.VMEM((EPW,), jnp.int32),
            pltpu.VMEM((EPW,), jnp.int32),
            pltpu.VMEM((C, D), jnp.float32),
            pltpu.VMEM((C, D), jnp.float32),
            pltpu.VMEM((C, D), jnp.float32),
            pltpu.VMEM((C, D), jnp.float32),
            pltpu.VMEM((EPW,), jnp.float32),
            pltpu.VMEM((L,), jnp.float32),
            pltpu.SemaphoreType.DMA,
            pltpu.SemaphoreType.DMA,
            pltpu.SemaphoreType.DMA,
        ],
    )
    return f(pn, cn, ei, bias16)


def kernel(patient_embeds, condition_embeds, edge_index, scale, bias):
    pn, cn = _normalize2(patient_embeds, condition_embeds, scale)
    ei = edge_index.astype(jnp.int32).reshape(2 * E)
    bias16 = jnp.broadcast_to(bias.astype(jnp.float32), (16,))
    return _sc_call(pn, cn, ei, bias16)


# speedup vs baseline: 1.1466x; 1.0515x over previous
import jax
import jax.numpy as jnp
from jax import lax
from jax.experimental import pallas as pl
from jax.experimental.pallas import tpu as pltpu
from jax.experimental.pallas import tpu_sc as plsc

N = 10000
D = 128
E = 320000
NC = 2
NS = 16
NW = NC * NS
L = 16
EPW = E // NW
C = 192
NCHUNK = EPW // C
TAIL = EPW - NCHUNK * C


def _norm_body(mult_ref, x1_ref, x2_ref, o1_ref, o2_ref):
    for x_ref, o_ref, m in ((x1_ref, o1_ref, 0), (x2_ref, o2_ref, 1)):
        x = x_ref[...]
        n2 = jnp.sum(x * x, axis=1, keepdims=True)
        r = jnp.maximum(jnp.sqrt(n2), 1e-8)
        o_ref[...] = x * (mult_ref[0, m] / r)


def _normalize2(t1, t2, scale):
    mult = jnp.stack([jnp.asarray(scale, jnp.float32),
                      jnp.float32(1.0)]).reshape(1, 2)
    blk = pl.BlockSpec((N // 10, D), lambda i: (i, 0))
    return pl.pallas_call(
        _norm_body,
        grid=(10,),
        in_specs=[pl.BlockSpec(memory_space=pltpu.SMEM), blk, blk],
        out_specs=[blk, blk],
        out_shape=[jax.ShapeDtypeStruct((N, D), jnp.float32)] * 2,
    )(mult, t1, t2)


def _sc_body(pn_hbm, cn_hbm, ei_hbm, bias_hbm, out_hbm,
             sidx, didx, srowsA, drowsA, srowsB, drowsB, outv, biasv,
             semA, semB, semI):
    cid = lax.axis_index("c")
    sid = lax.axis_index("s")
    wid = sid * NC + cid
    base = wid * EPW

    pltpu.sync_copy(bias_hbm, biasv)
    c1 = pltpu.async_copy(ei_hbm.at[pl.ds(base, EPW)], sidx, semI)
    c2 = pltpu.async_copy(ei_hbm.at[pl.ds(E + base, EPW)], didx, semI)
    c1.wait()
    c2.wait()
    bvec = biasv[...]

    def _issue(ci, srows, drows, sem):
        off = ci * C
        pltpu.async_copy(pn_hbm.at[sidx.at[pl.ds(off, C)]], srows, sem)
        pltpu.async_copy(cn_hbm.at[didx.at[pl.ds(off, C)]], drows, sem)

    def _wait(srows, drows, sem):
        pltpu.make_async_copy(pn_hbm.at[pl.ds(0, C)], srows, sem).wait()
        pltpu.make_async_copy(cn_hbm.at[pl.ds(0, C)], drows, sem).wait()

    lane = lax.iota(jnp.int32, L)

    def _dot_group(srows, drows, rows):
        def _k(k, acc):
            kv = (lane + k) & (D - 1)
            sv = plsc.load_gather(srows, [rows, kv])
            dv = plsc.load_gather(drows, [rows, kv])
            return acc + sv * dv

        return lax.fori_loop(0, D, _k, jnp.zeros((L,), jnp.float32),
                             unroll=16)

    def _compute(ci, srows, drows):
        @pl.loop(0, C // L)
        def _grp(g):
            acc = _dot_group(srows, drows, g * L + lane)
            outv[pl.ds(ci * C + g * L, L)] = acc + bvec

    _issue(0, srowsA, drowsA, semA)

    @pl.loop(0, (NCHUNK - 2) // 2)
    def _pair(i):
        ci = 1 + 2 * i
        _issue(ci, srowsB, drowsB, semB)
        _wait(srowsA, drowsA, semA)
        _compute(ci - 1, srowsA, drowsA)
        _issue(ci + 1, srowsA, drowsA, semA)
        _wait(srowsB, drowsB, semB)
        _compute(ci, srowsB, drowsB)

    _issue(NCHUNK - 1, srowsB, drowsB, semB)
    _wait(srowsA, drowsA, semA)
    _compute(NCHUNK - 2, srowsA, drowsA)
    toff = NCHUNK * C
    pltpu.async_copy(pn_hbm.at[sidx.at[pl.ds(toff, TAIL)]],
                     srowsA.at[pl.ds(0, TAIL), :], semA)
    pltpu.async_copy(cn_hbm.at[didx.at[pl.ds(toff, TAIL)]],
                     drowsA.at[pl.ds(0, TAIL), :], semA)
    _wait(srowsB, drowsB, semB)
    _compute(NCHUNK - 1, srowsB, drowsB)
    pltpu.make_async_copy(pn_hbm.at[pl.ds(0, TAIL)],
                          srowsA.at[pl.ds(0, TAIL), :], semA).wait()
    pltpu.make_async_copy(cn_hbm.at[pl.ds(0, TAIL)],
                          drowsA.at[pl.ds(0, TAIL), :], semA).wait()
    acc = _dot_group(srowsA, drowsA, lane)
    outv[pl.ds(toff, L)] = acc + bvec

    pltpu.sync_copy(outv, out_hbm.at[pl.ds(base, EPW)])


def _sc_call(pn, cn, ei, bias16):
    mesh = plsc.VectorSubcoreMesh(core_axis_name="c", subcore_axis_name="s",
                                  num_cores=NC, num_subcores=NS)
    f = pl.kernel(
        _sc_body,
        out_type=jax.ShapeDtypeStruct((E,), jnp.float32),
        mesh=mesh,
        compiler_params=pltpu.CompilerParams(needs_layout_passes=False),
        scratch_types=[
            pltpu.VMEM((EPW,), jnp.int32),
            pltpu.VMEM((EPW,), jnp.int32),
            pltpu.VMEM((C, D), jnp.float32),
            pltpu.VMEM((C, D), jnp.float32),
            pltpu.VMEM((C, D), jnp.float32),
            pltpu.VMEM((C, D), jnp.float32),
            pltpu.VMEM((EPW,), jnp.float32),
            pltpu.VMEM((L,), jnp.float32),
            pltpu.SemaphoreType.DMA,
            pltpu.SemaphoreType.DMA,
            pltpu.SemaphoreType.DMA,
        ],
    )
    return f(pn, cn, ei, bias16)


def kernel(patient_embeds, condition_embeds, edge_index, scale, bias):
    pn, cn = _normalize2(patient_embeds, condition_embeds, scale)
    ei = edge_index.astype(jnp.int32).reshape(2 * E)
    bias16 = jnp.broadcast_to(bias.astype(jnp.float32), (16,))
    return _sc_call(pn, cn, ei, bias16)
